# baseline (speedup 1.0000x reference)
import jax
import jax.numpy as jnp
from jax.experimental import pallas as pl
from jax.experimental.pallas import tpu as pltpu

_N = 16384
_D = 1024
_CHUNKS = 8
_ROWS = _N // _CHUNKS


def _copy_body(x_hbm, o_hbm, *sems):
    copies = []
    for c in range(_CHUNKS):
        rows = pl.ds(c * _ROWS, _ROWS)
        copies.append(
            pltpu.make_async_copy(x_hbm.at[rows, :], o_hbm.at[rows, :], sems[c])
        )
    for cp in copies:
        cp.start()
    for cp in copies:
        cp.wait()


def kernel(input):
    return pl.pallas_call(
        _copy_body,
        in_specs=[pl.BlockSpec(memory_space=pltpu.MemorySpace.HBM)],
        out_specs=pl.BlockSpec(memory_space=pltpu.MemorySpace.HBM),
        out_shape=jax.ShapeDtypeStruct((_N, _D), jnp.float32),
        scratch_shapes=[pltpu.SemaphoreType.DMA] * _CHUNKS,
    )(input)


# speedup vs baseline: 10.3335x; 10.3335x over previous
import jax
import jax.numpy as jnp
from jax.experimental import pallas as pl
from jax.experimental.pallas import tpu as pltpu

_N = 16384
_D = 1024
_WIDE = 8192
_NW = (_N * _D) // _WIDE
_BLOCK_ROWS = 256


def _copy_body(x_ref, o_ref):
    o_ref[...] = x_ref[...]


def kernel(input):
    x = input.reshape(_NW, _WIDE)
    out = pl.pallas_call(
        _copy_body,
        grid=(_NW // _BLOCK_ROWS,),
        in_specs=[pl.BlockSpec((_BLOCK_ROWS, _WIDE), lambda i: (i, 0))],
        out_specs=pl.BlockSpec((_BLOCK_ROWS, _WIDE), lambda i: (i, 0)),
        out_shape=jax.ShapeDtypeStruct((_NW, _WIDE), jnp.float32),
        compiler_params=pltpu.CompilerParams(
            dimension_semantics=("arbitrary",),
        ),
    )(x)
    return out.reshape(_N, _D)


# speedup vs baseline: 48.8513x; 4.7275x over previous
import jax
import jax.numpy as jnp
from jax.experimental import pallas as pl
from jax.experimental.pallas import tpu as pltpu

_N = 16384
_D = 1024
_BLOCK_ROWS = 2048


def _copy_body(x_ref, o_ref):
    o_ref[...] = x_ref[...]


def kernel(input):
    return pl.pallas_call(
        _copy_body,
        grid=(_N // _BLOCK_ROWS,),
        in_specs=[pl.BlockSpec((_BLOCK_ROWS, _D), lambda i: (i, 0))],
        out_specs=pl.BlockSpec((_BLOCK_ROWS, _D), lambda i: (i, 0)),
        out_shape=jax.ShapeDtypeStruct((_N, _D), jnp.float32),
        compiler_params=pltpu.CompilerParams(
            dimension_semantics=("arbitrary",),
        ),
    )(input)
